# Initial kernel scaffold; baseline (speedup 1.0000x reference)
import jax
import jax.numpy as jnp
from jax.experimental import pallas as pl


def kernel(input_features, coords):
    raise NotImplementedError("write your pallas kernel here")


# speedup vs baseline: 2.4617x; 2.4617x over previous
import jax
import jax.numpy as jnp
from jax import lax
from jax.experimental import pallas as pl
from jax.experimental.pallas import tpu as pltpu
from jax.experimental.pallas import tpu_sc as plsc

N = 1_000_000
C = 32
HALF = 16
OUT_SIZE = 64
S = OUT_SIZE ** 3
SCALE = 0.125

NS = 16
L = 16

K = 512
ROWS_MAIN = 62_464
CHUNKS_MAIN = ROWS_MAIN // K
ROWS_LAST = N - (NS - 1) * ROWS_MAIN
TAIL = ROWS_LAST - 123 * K

DUMP = 4352
RSIZES = (98_304, 98_304, 65_536)
ROFFS = (0, 98_304, 196_608)
BUF_ROWS = DUMP + RSIZES[0]


def _body(feat, ct, out, c0buf, c1buf, c2buf, fbuf,
          ibuf0, ibuf1, ibuf2, ibuf3, ibuf_t, obuf, zbuf,
          spmem, gsem, ssem, osem):
    core = lax.axis_index("c")
    tile = lax.axis_index("s")
    base_row = tile * ROWS_MAIN
    col0 = core * HALF
    iota = lax.iota(jnp.int32, L)
    zeros16 = jnp.zeros((L,), jnp.float32)

    def _zb(t, _):
        zbuf[t, :] = zeros16
        return _
    lax.fori_loop(0, 128, _zb, None)

    nchunks = jnp.where(tile == NS - 1, 123, CHUNKS_MAIN)

    def sweep(pidx, roff, rsize):
        rbase = roff
        sl = rsize // NS
        my0 = DUMP + tile * sl

        def _zero(k, _):
            pltpu.sync_copy(zbuf, spmem.at[pl.ds(my0 + k * 128, 128), :])
            return _
        lax.fori_loop(0, sl // 128, _zero, None)
        plsc.subcore_barrier()

        def chunk(hoff, nrows, idxrefs):
            waits = []
            for d, cb in enumerate((c0buf, c1buf, c2buf)):
                waits.append(pltpu.async_copy(
                    ct.at[pl.ds(d * N + hoff, nrows)],
                    cb.at[pl.ds(0, nrows)], gsem))
            waits.append(pltpu.async_copy(
                feat.at[pl.ds(hoff, nrows), pl.ds(col0, HALF)],
                fbuf.at[pl.ds(0, nrows), :], gsem))
            for w in waits:
                w.wait()
            for q in range(0, nrows, L):
                c0 = c0buf[pl.ds(q, L)]
                c1 = c1buf[pl.ds(q, L)]
                c2 = c2buf[pl.ds(q, L)]
                seg = ((c0 >> 1) << 12) + ((c1 >> 1) << 6) + (c2 >> 1)
                d = seg - rbase
                ok = (d >= 0) & (d < rsize)
                dump = (tile * 256 + (q % 256)) + iota
                idx = jnp.where(ok, d + DUMP, dump)
                idxrefs[q // 128][pl.ds(q % 128, L)] = idx
            sc_waits = []
            for j in range(nrows // 128):
                sc_waits.append(pltpu.async_copy(
                    fbuf.at[pl.ds(j * 128, 128), :],
                    spmem.at[idxrefs[j]], ssem, add=True))
            for w in sc_waits:
                w.wait()

        def _chunk_body(i, _):
            chunk(base_row + i * K, K, (ibuf0, ibuf1, ibuf2, ibuf3))
            return _
        lax.fori_loop(0, nchunks, _chunk_body, None)

        @pl.when(tile == NS - 1)
        def _tail():
            hoff = base_row + 123 * K
            w0 = pltpu.async_copy(
                feat.at[pl.ds(hoff, TAIL), pl.ds(col0, HALF)],
                fbuf.at[pl.ds(0, TAIL), :], gsem)
            wc = [pltpu.async_copy(
                ct.at[pl.ds(d * N + hoff, TAIL)],
                cb.at[pl.ds(0, TAIL)], gsem)
                for d, cb in enumerate((c0buf, c1buf, c2buf))]
            for w in wc:
                w.wait()
            w0.wait()
            for q in range(0, TAIL, L):
                c0 = c0buf[pl.ds(q, L)]
                c1 = c1buf[pl.ds(q, L)]
                c2 = c2buf[pl.ds(q, L)]
                seg = ((c0 >> 1) << 12) + ((c1 >> 1) << 6) + (c2 >> 1)
                d = seg - rbase
                ok = (d >= 0) & (d < rsize)
                dump = (tile * 256 + (q % 256)) + iota
                idx = jnp.where(ok, d + DUMP, dump)
                ibuf_t[pl.ds(q, L)] = idx
            pltpu.async_copy(
                fbuf.at[pl.ds(0, TAIL), :],
                spmem.at[ibuf_t], ssem, add=True).wait()

        plsc.subcore_barrier()

        def _copyout(k, _):
            src0 = my0 + k * K
            pltpu.sync_copy(spmem.at[pl.ds(src0, K), :], obuf)

            def _scale(j, _2):
                for t in range(16):
                    row = j * 16 + t
                    obuf[row, :] = obuf[row, :] * SCALE
                return _2
            lax.fori_loop(0, K // 16, _scale, None)
            orow = rbase + tile * sl + k * K
            pltpu.async_copy(
                obuf, out.at[pl.ds(orow, K), pl.ds(col0, HALF)], osem).wait()
            return _
        lax.fori_loop(0, sl // K, _copyout, None)

    for pidx in range(3):
        sweep(pidx, ROFFS[pidx], RSIZES[pidx])


def kernel(input_features, coords):
    ct = coords.T.reshape(3 * N)
    fn = pl.kernel(
        _body,
        out_type=jax.ShapeDtypeStruct((S, C), jnp.float32),
        mesh=plsc.VectorSubcoreMesh(core_axis_name="c", subcore_axis_name="s"),
        compiler_params=pltpu.CompilerParams(use_tc_tiling_on_sc=False),
        scratch_types=[
            pltpu.VMEM((K,), jnp.int32),
            pltpu.VMEM((K,), jnp.int32),
            pltpu.VMEM((K,), jnp.int32),
            pltpu.VMEM((K, HALF), jnp.float32),
            pltpu.VMEM((128,), jnp.int32),
            pltpu.VMEM((128,), jnp.int32),
            pltpu.VMEM((128,), jnp.int32),
            pltpu.VMEM((128,), jnp.int32),
            pltpu.VMEM((TAIL,), jnp.int32),
            pltpu.VMEM((K, HALF), jnp.float32),
            pltpu.VMEM((128, HALF), jnp.float32),
            pltpu.VMEM_SHARED((BUF_ROWS, HALF), jnp.float32),
            pltpu.SemaphoreType.DMA,
            pltpu.SemaphoreType.DMA,
            pltpu.SemaphoreType.DMA,
        ],
    )
    return fn(input_features, ct)


# speedup vs baseline: 2.9678x; 1.2056x over previous
import jax
import jax.numpy as jnp
from jax import lax
from jax.experimental import pallas as pl
from jax.experimental.pallas import tpu as pltpu
from jax.experimental.pallas import tpu_sc as plsc

N = 1_000_000
C = 32
HALF = 16
OUT_SIZE = 64
S = OUT_SIZE ** 3
SCALE = 0.125

NS = 16
L = 16

K = 256
ROWS_MAIN = 62_464
CHUNKS_MAIN = ROWS_MAIN // K
ROWS_LAST = N - (NS - 1) * ROWS_MAIN
TAIL = ROWS_LAST - 246 * K

DUMP = 4352
RSIZE = 98_304
NSWEEP = 3


def _body(feat, ct, out,
          c0b, c1b, c2b, fb, ib, ibuf_t, obuf, zbuf, spmem,
          gs0, gs1, gs2, gs3, ss0, ss1, ss2, ss3, osem):
    core = lax.axis_index("c")
    tile = lax.axis_index("s")
    base_row = tile * ROWS_MAIN
    col0 = core * HALF
    iota = lax.iota(jnp.int32, L)
    zeros16 = jnp.zeros((L,), jnp.float32)
    gsem = (gs0, gs1, gs2, gs3)
    ssem = (ss0, ss1, ss2, ss3)

    def _zb(t, _):
        zbuf[t, :] = zeros16
        return _
    lax.fori_loop(0, 64, _zb, None)

    def fire_gather(s, c):
        hoff = base_row + c * K
        pltpu.async_copy(ct.at[pl.ds(hoff, K)], c0b.at[s], gsem[s])
        pltpu.async_copy(ct.at[pl.ds(N + hoff, K)], c1b.at[s], gsem[s])
        pltpu.async_copy(ct.at[pl.ds(2 * N + hoff, K)], c2b.at[s], gsem[s])
        pltpu.async_copy(feat.at[pl.ds(hoff, K), pl.ds(col0, HALF)],
                         fb.at[s], gsem[s])

    def wait_gather(s):
        pltpu.make_async_copy(ct.at[pl.ds(0, K)], c0b.at[s], gsem[s]).wait()
        pltpu.make_async_copy(ct.at[pl.ds(0, K)], c1b.at[s], gsem[s]).wait()
        pltpu.make_async_copy(ct.at[pl.ds(0, K)], c2b.at[s], gsem[s]).wait()
        pltpu.make_async_copy(feat.at[pl.ds(0, K), pl.ds(col0, HALF)],
                              fb.at[s], gsem[s]).wait()

    def fire_scatter(s):
        pltpu.async_copy(fb.at[s, pl.ds(0, 128), :],
                         spmem.at[ib.at[2 * s]], ssem[s], add=True)
        pltpu.async_copy(fb.at[s, pl.ds(128, 128), :],
                         spmem.at[ib.at[2 * s + 1]], ssem[s], add=True)

    def wait_scatter(s):
        pltpu.make_async_copy(fb.at[s, pl.ds(0, 128), :],
                              spmem.at[ib.at[2 * s]], ssem[s]).wait()
        pltpu.make_async_copy(fb.at[s, pl.ds(128, 128), :],
                              spmem.at[ib.at[2 * s + 1]], ssem[s]).wait()

    def build_idx(s, rbase):
        for q in range(0, K, L):
            c0 = c0b[s, pl.ds(q, L)]
            c1 = c1b[s, pl.ds(q, L)]
            c2 = c2b[s, pl.ds(q, L)]
            seg = ((c0 >> 1) << 12) + ((c1 >> 1) << 6) + (c2 >> 1)
            d = seg - rbase
            ok = (d >= 0) & (d < RSIZE)
            dump = (tile * 256 + (q % 256)) + iota
            idx = jnp.where(ok, d + DUMP, dump)
            ib[2 * s + q // 128, pl.ds(q % 128, L)] = idx

    def sweep(p, _):
        rbase = p * RSIZE
        last = p == NSWEEP - 1
        sl = jnp.where(last, 4096, 6144)
        my0 = DUMP + tile * sl

        def _zero(k, _2):
            pltpu.sync_copy(zbuf, spmem.at[pl.ds(my0 + k * 64, 64), :])
            return _2
        lax.fori_loop(0, jnp.where(last, 64, 96), _zero, None)
        plsc.subcore_barrier()

        fire_gather(0, 0)
        fire_gather(1, 1)

        def quad(i, _2):
            for s in range(4):
                c = 4 * i + s
                wait_gather(s)
                build_idx(s, rbase)
                fire_scatter(s)
                t = (s + 2) % 4
                if s < 2:
                    @pl.when(i > 0)
                    def _w():
                        wait_scatter(t)
                else:
                    wait_scatter(t)
                fire_gather(t, c + 2)
            return _2
        lax.fori_loop(0, CHUNKS_MAIN // 4, quad, None)

        wait_gather(0)
        wait_gather(1)
        wait_scatter(2)
        wait_scatter(3)

        @pl.when(tile == NS - 1)
        def _extra():
            build_idx(0, rbase)
            fire_scatter(0)
            build_idx(1, rbase)
            fire_scatter(1)
            wait_scatter(0)
            wait_scatter(1)
            hoff = base_row + 246 * K
            pltpu.async_copy(ct.at[pl.ds(hoff, TAIL)],
                             c0b.at[2, pl.ds(0, TAIL)], gs2)
            pltpu.async_copy(ct.at[pl.ds(N + hoff, TAIL)],
                             c1b.at[2, pl.ds(0, TAIL)], gs2)
            pltpu.async_copy(ct.at[pl.ds(2 * N + hoff, TAIL)],
                             c2b.at[2, pl.ds(0, TAIL)], gs2)
            pltpu.async_copy(feat.at[pl.ds(hoff, TAIL), pl.ds(col0, HALF)],
                             fb.at[2, pl.ds(0, TAIL), :], gs2)
            pltpu.make_async_copy(ct.at[pl.ds(0, TAIL)],
                                  c0b.at[2, pl.ds(0, TAIL)], gs2).wait()
            pltpu.make_async_copy(ct.at[pl.ds(0, TAIL)],
                                  c1b.at[2, pl.ds(0, TAIL)], gs2).wait()
            pltpu.make_async_copy(ct.at[pl.ds(0, TAIL)],
                                  c2b.at[2, pl.ds(0, TAIL)], gs2).wait()
            pltpu.make_async_copy(feat.at[pl.ds(0, TAIL), pl.ds(col0, HALF)],
                                  fb.at[2, pl.ds(0, TAIL), :], gs2).wait()
            for q in range(0, TAIL, L):
                c0 = c0b[2, pl.ds(q, L)]
                c1 = c1b[2, pl.ds(q, L)]
                c2 = c2b[2, pl.ds(q, L)]
                seg = ((c0 >> 1) << 12) + ((c1 >> 1) << 6) + (c2 >> 1)
                d = seg - rbase
                ok = (d >= 0) & (d < RSIZE)
                dump = (tile * 256 + (q % 256)) + iota
                idx = jnp.where(ok, d + DUMP, dump)
                ibuf_t[pl.ds(q, L)] = idx
            pltpu.async_copy(fb.at[2, pl.ds(0, TAIL), :],
                             spmem.at[ibuf_t], ss2, add=True)
            pltpu.make_async_copy(fb.at[2, pl.ds(0, TAIL), :],
                                  spmem.at[ibuf_t], ss2).wait()

        plsc.subcore_barrier()

        def _copyout(k, _2):
            src0 = my0 + k * 256
            pltpu.sync_copy(spmem.at[pl.ds(src0, 256), :], obuf)

            def _scale(j, _3):
                for t in range(16):
                    row = j * 16 + t
                    obuf[row, :] = obuf[row, :] * SCALE
                return _3
            lax.fori_loop(0, 16, _scale, None)
            orow = rbase + tile * sl + k * 256
            pltpu.async_copy(
                obuf, out.at[pl.ds(orow, 256), pl.ds(col0, HALF)], osem).wait()
            return _2
        lax.fori_loop(0, jnp.where(last, 16, 24), _copyout, None)
        return _

    lax.fori_loop(0, NSWEEP, sweep, None)


def kernel(input_features, coords):
    ct = coords.T.reshape(3 * N)
    fn = pl.kernel(
        _body,
        out_type=jax.ShapeDtypeStruct((S, C), jnp.float32),
        mesh=plsc.VectorSubcoreMesh(core_axis_name="c", subcore_axis_name="s"),
        compiler_params=pltpu.CompilerParams(use_tc_tiling_on_sc=False),
        scratch_types=[
            pltpu.VMEM((4, K), jnp.int32),
            pltpu.VMEM((4, K), jnp.int32),
            pltpu.VMEM((4, K), jnp.int32),
            pltpu.VMEM((4, K, HALF), jnp.float32),
            pltpu.VMEM((8, 128), jnp.int32),
            pltpu.VMEM((TAIL,), jnp.int32),
            pltpu.VMEM((256, HALF), jnp.float32),
            pltpu.VMEM((64, HALF), jnp.float32),
            pltpu.VMEM_SHARED((DUMP + RSIZE, HALF), jnp.float32),
            pltpu.SemaphoreType.DMA,
            pltpu.SemaphoreType.DMA,
            pltpu.SemaphoreType.DMA,
            pltpu.SemaphoreType.DMA,
            pltpu.SemaphoreType.DMA,
            pltpu.SemaphoreType.DMA,
            pltpu.SemaphoreType.DMA,
            pltpu.SemaphoreType.DMA,
            pltpu.SemaphoreType.DMA,
        ],
    )
    return fn(input_features, ct)


# speedup vs baseline: 3.0171x; 1.0166x over previous
import jax
import jax.numpy as jnp
from jax import lax
from jax.experimental import pallas as pl
from jax.experimental.pallas import tpu as pltpu
from jax.experimental.pallas import tpu_sc as plsc

N = 1_000_000
C = 32
HALF = 16
OUT_SIZE = 64
S = OUT_SIZE ** 3
SCALE = 0.125

NS = 16
L = 16

K = 256
ROWS_MAIN = 62_464
CHUNKS_MAIN = ROWS_MAIN // K
ROWS_LAST = N - (NS - 1) * ROWS_MAIN
TAIL = ROWS_LAST - 246 * K

DUMP = 4352
RSIZE = 98_304
NSWEEP = 3


def _body(feat, sg, out,
          sgb, fb, ib, ibuf_t, obuf, zbuf, spmem,
          gs0, gs1, gs2, gs3, ss0, ss1, ss2, ss3, osem):
    core = lax.axis_index("c")
    tile = lax.axis_index("s")
    base_row = tile * ROWS_MAIN
    col0 = core * HALF
    iota = lax.iota(jnp.int32, L)
    zeros16 = jnp.zeros((L,), jnp.float32)
    gsem = (gs0, gs1, gs2, gs3)
    ssem = (ss0, ss1, ss2, ss3)

    def _zb(t, _):
        zbuf[t, :] = zeros16
        return _
    lax.fori_loop(0, 64, _zb, None)

    def fire_gather(s, c):
        hoff = base_row + c * K
        pltpu.async_copy(sg.at[pl.ds(hoff, K)], sgb.at[s], gsem[s])
        pltpu.async_copy(feat.at[pl.ds(hoff, K), pl.ds(col0, HALF)],
                         fb.at[s], gsem[s])

    def wait_gather(s):
        pltpu.make_async_copy(sg.at[pl.ds(0, K)], sgb.at[s], gsem[s]).wait()
        pltpu.make_async_copy(feat.at[pl.ds(0, K), pl.ds(col0, HALF)],
                              fb.at[s], gsem[s]).wait()

    def fire_scatter(s):
        pltpu.async_copy(fb.at[s, pl.ds(0, 128), :],
                         spmem.at[ib.at[2 * s]], ssem[s], add=True)
        pltpu.async_copy(fb.at[s, pl.ds(128, 128), :],
                         spmem.at[ib.at[2 * s + 1]], ssem[s], add=True)

    def wait_scatter(s):
        pltpu.make_async_copy(fb.at[s, pl.ds(0, 128), :],
                              spmem.at[ib.at[2 * s]], ssem[s]).wait()
        pltpu.make_async_copy(fb.at[s, pl.ds(128, 128), :],
                              spmem.at[ib.at[2 * s + 1]], ssem[s]).wait()

    def build_idx(s, rbase):
        for q in range(0, K, L):
            seg = sgb[s, pl.ds(q, L)]
            d = seg - rbase
            ok = (d >= 0) & (d < RSIZE)
            dump = (tile * 256 + (q % 256)) + iota
            idx = jnp.where(ok, d + DUMP, dump)
            ib[2 * s + q // 128, pl.ds(q % 128, L)] = idx

    def sweep(p, _):
        rbase = p * RSIZE
        last = p == NSWEEP - 1
        sl = jnp.where(last, 4096, 6144)
        my0 = DUMP + tile * sl

        def _zero(k, _2):
            pltpu.sync_copy(zbuf, spmem.at[pl.ds(my0 + k * 64, 64), :])
            return _2
        lax.fori_loop(0, jnp.where(last, 64, 96), _zero, None)
        plsc.subcore_barrier()

        fire_gather(0, 0)
        fire_gather(1, 1)

        def quad(i, _2):
            for s in range(4):
                c = 4 * i + s
                wait_gather(s)
                build_idx(s, rbase)
                fire_scatter(s)
                t = (s + 2) % 4
                if s < 2:
                    @pl.when(i > 0)
                    def _w():
                        wait_scatter(t)
                else:
                    wait_scatter(t)
                fire_gather(t, c + 2)
            return _2
        lax.fori_loop(0, CHUNKS_MAIN // 4, quad, None)

        wait_gather(0)
        wait_gather(1)
        wait_scatter(2)
        wait_scatter(3)

        @pl.when(tile == NS - 1)
        def _extra():
            build_idx(0, rbase)
            fire_scatter(0)
            build_idx(1, rbase)
            fire_scatter(1)
            wait_scatter(0)
            wait_scatter(1)
            hoff = base_row + 246 * K
            pltpu.async_copy(sg.at[pl.ds(hoff, TAIL)],
                             sgb.at[2, pl.ds(0, TAIL)], gs2)
            pltpu.async_copy(feat.at[pl.ds(hoff, TAIL), pl.ds(col0, HALF)],
                             fb.at[2, pl.ds(0, TAIL), :], gs2)
            pltpu.make_async_copy(sg.at[pl.ds(0, TAIL)],
                                  sgb.at[2, pl.ds(0, TAIL)], gs2).wait()
            pltpu.make_async_copy(feat.at[pl.ds(0, TAIL), pl.ds(col0, HALF)],
                                  fb.at[2, pl.ds(0, TAIL), :], gs2).wait()
            for q in range(0, TAIL, L):
                seg = sgb[2, pl.ds(q, L)]
                d = seg - rbase
                ok = (d >= 0) & (d < RSIZE)
                dump = (tile * 256 + (q % 256)) + iota
                idx = jnp.where(ok, d + DUMP, dump)
                ibuf_t[pl.ds(q, L)] = idx
            pltpu.async_copy(fb.at[2, pl.ds(0, TAIL), :],
                             spmem.at[ibuf_t], ss2, add=True)
            pltpu.make_async_copy(fb.at[2, pl.ds(0, TAIL), :],
                                  spmem.at[ibuf_t], ss2).wait()

        plsc.subcore_barrier()

        def _copyout(k, _2):
            src0 = my0 + k * 256
            pltpu.sync_copy(spmem.at[pl.ds(src0, 256), :], obuf)

            def _scale(j, _3):
                for t in range(16):
                    row = j * 16 + t
                    obuf[row, :] = obuf[row, :] * SCALE
                return _3
            lax.fori_loop(0, 16, _scale, None)
            orow = rbase + tile * sl + k * 256
            pltpu.async_copy(
                obuf, out.at[pl.ds(orow, 256), pl.ds(col0, HALF)], osem).wait()
            return _2
        lax.fori_loop(0, jnp.where(last, 16, 24), _copyout, None)
        return _

    lax.fori_loop(0, NSWEEP, sweep, None)


def kernel(input_features, coords):
    seg = (((coords[:, 0] >> 1) << 12)
           + ((coords[:, 1] >> 1) << 6)
           + (coords[:, 2] >> 1)).astype(jnp.int32)
    fn = pl.kernel(
        _body,
        out_type=jax.ShapeDtypeStruct((S, C), jnp.float32),
        mesh=plsc.VectorSubcoreMesh(core_axis_name="c", subcore_axis_name="s"),
        compiler_params=pltpu.CompilerParams(use_tc_tiling_on_sc=False),
        scratch_types=[
            pltpu.VMEM((4, K), jnp.int32),
            pltpu.VMEM((4, K, HALF), jnp.float32),
            pltpu.VMEM((8, 128), jnp.int32),
            pltpu.VMEM((TAIL,), jnp.int32),
            pltpu.VMEM((256, HALF), jnp.float32),
            pltpu.VMEM((64, HALF), jnp.float32),
            pltpu.VMEM_SHARED((DUMP + RSIZE, HALF), jnp.float32),
            pltpu.SemaphoreType.DMA,
            pltpu.SemaphoreType.DMA,
            pltpu.SemaphoreType.DMA,
            pltpu.SemaphoreType.DMA,
            pltpu.SemaphoreType.DMA,
            pltpu.SemaphoreType.DMA,
            pltpu.SemaphoreType.DMA,
            pltpu.SemaphoreType.DMA,
            pltpu.SemaphoreType.DMA,
        ],
    )
    return fn(input_features, seg)
